# Initial kernel scaffold; baseline (speedup 1.0000x reference)
import jax
import jax.numpy as jnp
from jax.experimental import pallas as pl


def kernel(boxes, scores):
    raise NotImplementedError("write your pallas kernel here")


# speedup vs baseline: 39.2970x; 39.2970x over previous
import jax
import jax.numpy as jnp
from jax.experimental import pallas as pl
from jax.experimental.pallas import tpu as pltpu

N = 20000
K = 2000
KPAD = 2048
BLK = 128
NBLK = KPAD // BLK
MAXOUT = 500
OUTPAD = 512
THR2 = 1.0
HALF_IMG = 256.0
_HIGH = jax.lax.Precision.HIGHEST


def _nms_body(scores_row_ref, scores_col_ref, boxes_ref, boxesT_ref, out_ref,
              cxr, cyr, cxc, cyc, s_row_ref, s_col_ref, keep_ref):
    f32 = jnp.float32
    cxr[...] = (boxesT_ref[0:1, :] + boxesT_ref[2:3, :]) * HALF_IMG
    cyr[...] = (boxesT_ref[1:2, :] + boxesT_ref[3:4, :]) * HALF_IMG
    cxc[...] = (boxes_ref[:, 0:1] + boxes_ref[:, 2:3]) * HALF_IMG
    cyc[...] = (boxes_ref[:, 1:2] + boxes_ref[:, 3:4]) * HALF_IMG

    lane = jax.lax.broadcasted_iota(jnp.int32, (1, KPAD), 1)
    s0_row = jnp.where(lane >= K, 1.0, 0.0).astype(f32)
    s_row_ref[...] = s0_row
    lane_c = jax.lax.broadcasted_iota(jnp.int32, (KPAD, 1), 0)
    s_col_ref[...] = jnp.where(lane_c >= K, 1.0, 0.0).astype(f32)
    keep_ref[...] = jnp.zeros((1, KPAD), f32)

    ir = jax.lax.broadcasted_iota(jnp.int32, (BLK, BLK), 0)
    ic = jax.lax.broadcasted_iota(jnp.int32, (BLK, BLK), 1)
    upper = (ir < ic).astype(f32)
    lower = (ic < ir).astype(f32)

    def block_step(k, _):
        total = jnp.sum(keep_ref[...])

        @pl.when(total < MAXOUT)
        def _():
            base = pl.multiple_of(k * BLK, BLK)
            bcx = cxc[pl.ds(base, BLK), :]
            bcy = cyc[pl.ds(base, BLK), :]
            rcx = cxr[:, pl.ds(base, BLK)]
            rcy = cyr[:, pl.ds(base, BLK)]
            d2b = (bcx - rcx) ** 2 + (bcy - rcy) ** 2
            csym = (d2b < THR2).astype(f32)
            cu = csym * upper
            cl = csym * lower
            se_row = s_row_ref[:, pl.ds(base, BLK)]
            se_col = s_col_ref[pl.ds(base, BLK), :]

            def cond(c):
                _, _, it, changed = c
                return jnp.logical_and(it < BLK + 4, changed)

            def body(c):
                sr, sc, it, _ = c
                alive_r = 1.0 - sr
                ncol = jnp.max(cl * alive_r, axis=1, keepdims=True)
                sc2 = jnp.maximum(se_col, ncol)
                alive_c = 1.0 - sc2
                nrow = jnp.max(cu * alive_c, axis=0, keepdims=True)
                sr2 = jnp.maximum(se_row, nrow)
                ch = jnp.logical_or(jnp.any(sr2 != sr), jnp.any(sc2 != sc))
                return sr2, sc2, it + 1, ch

            sr_f, sc_f, _, _ = jax.lax.while_loop(
                cond, body, (se_row, se_col, 0, jnp.bool_(True)))
            keep_blk = 1.0 - sr_f
            keep_ref[:, pl.ds(base, BLK)] = keep_blk
            alive_c = 1.0 - sc_f

            d2r = (bcx - cxr[...]) ** 2 + (bcy - cyr[...]) ** 2
            hit_r = jnp.max((d2r < THR2).astype(f32) * alive_c,
                            axis=0, keepdims=True)
            s_row_ref[...] = jnp.maximum(s_row_ref[...], hit_r)
            d2c = (cxc[...] - rcx) ** 2 + (cyc[...] - rcy) ** 2
            kr = keep_blk
            hit_c = jnp.max((d2c < THR2).astype(f32) * kr,
                            axis=1, keepdims=True)
            s_col_ref[...] = jnp.maximum(s_col_ref[...], hit_c)
        return 0

    jax.lax.fori_loop(0, NBLK, block_step, 0)

    keep_row = keep_ref[...]
    jr = jax.lax.broadcasted_iota(jnp.int32, (KPAD, KPAD), 0)
    jc = jax.lax.broadcasted_iota(jnp.int32, (KPAD, KPAD), 1)
    ltri = (jr < jc).astype(f32)
    rank = jax.lax.dot_general(keep_row, ltri, (((1,), (0,)), ((), ())),
                               precision=_HIGH,
                               preferred_element_type=f32)
    out_iota = jax.lax.broadcasted_iota(jnp.int32, (OUTPAD, 1), 0).astype(f32)
    q = jnp.where(jnp.logical_and(rank == out_iota, keep_row > 0.5),
                  1.0, 0.0).astype(f32)
    data = jnp.concatenate([boxes_ref[...], scores_col_ref[...]],
                           axis=1)
    out = jax.lax.dot_general(q, data, (((1,), (0,)), ((), ())),
                              precision=_HIGH,
                              preferred_element_type=f32)
    total = jnp.sum(keep_row)
    row0 = out[0:1, :]
    pad_mask = (out_iota >= total).astype(f32)
    out = out + pad_mask * row0
    out_ref[...] = out[:MAXOUT, :]


def kernel(boxes, scores):
    topk_scores, topk_idx = jax.lax.top_k(scores, K)
    sel_boxes = jnp.take(boxes, topk_idx, axis=0)

    pad = KPAD - K
    ps = jnp.concatenate([topk_scores, jnp.full((pad,), -1.0, jnp.float32)])
    pb = jnp.concatenate(
        [sel_boxes, jnp.full((pad, 4), 1e6, jnp.float32)], axis=0)

    scores_row = ps[None, :]
    scores_col = ps[:, None]
    boxesT = pb.T

    return pl.pallas_call(
        _nms_body,
        out_shape=jax.ShapeDtypeStruct((MAXOUT, 5), jnp.float32),
        scratch_shapes=[
            pltpu.VMEM((1, KPAD), jnp.float32),
            pltpu.VMEM((1, KPAD), jnp.float32),
            pltpu.VMEM((KPAD, 1), jnp.float32),
            pltpu.VMEM((KPAD, 1), jnp.float32),
            pltpu.VMEM((1, KPAD), jnp.float32),
            pltpu.VMEM((KPAD, 1), jnp.float32),
            pltpu.VMEM((1, KPAD), jnp.float32),
        ],
    )(scores_row, scores_col, pb, boxesT)
